# baseline (speedup 1.0000x reference)
import functools

import jax
import jax.numpy as jnp
from jax import lax
from jax.experimental import pallas as pl
from jax.experimental.pallas import tpu as pltpu
from jax.experimental.pallas import tpu_sc as plsc

_EMBED = 64
_BATCH = 16384
_NC = 2
_NS = 16
_NW = _NC * _NS
_BPW = _BATCH // _NW
_CHUNK = 128
_NCHUNK = _BPW // _CHUNK
_GROUPS = _BPW // 16


def _sc_body(uidx_hbm, pidx_hbm, nidx_hbm, utab_hbm, itab_hbm,
             pos_hbm, neg_hbm,
             uidx_v, pidx_v, nidx_v, urows, prows, nrows,
             pos_v, neg_v, sem):
    c = lax.axis_index("c")
    s = lax.axis_index("s")
    wid = s * _NC + c
    base_chunk = wid * _NCHUNK

    pltpu.sync_copy(uidx_hbm.at[pl.ds(base_chunk, _NCHUNK)], uidx_v)
    pltpu.sync_copy(pidx_hbm.at[pl.ds(base_chunk, _NCHUNK)], pidx_v)
    pltpu.sync_copy(nidx_hbm.at[pl.ds(base_chunk, _NCHUNK)], nidx_v)

    copies = []
    for tab, idx_v, rows in ((utab_hbm, uidx_v, urows),
                             (itab_hbm, pidx_v, prows),
                             (itab_hbm, nidx_v, nrows)):
        for j in range(_NCHUNK):
            copies.append(
                pltpu.async_copy(tab.at[idx_v.at[j]],
                                 rows.at[pl.ds(j * _CHUNK, _CHUNK)], sem))
    for cp in copies:
        cp.wait()

    lane = lax.iota(jnp.int32, 16)

    def group_body(g, _):
        rows16 = g * 16 + lane
        accp = jnp.zeros((16,), jnp.float32)
        accn = jnp.zeros((16,), jnp.float32)
        for d in range(_EMBED):
            col = jnp.full((16,), d, jnp.int32)
            u = plsc.load_gather(urows, [rows16, col])
            p = plsc.load_gather(prows, [rows16, col])
            n = plsc.load_gather(nrows, [rows16, col])
            accp = accp + u * p
            accn = accn + u * n
        pos_v[pl.ds(g * 16, 16)] = accp
        neg_v[pl.ds(g * 16, 16)] = accn
        return 0

    lax.fori_loop(0, _GROUPS, group_body, 0)

    out_base = wid * _BPW
    pltpu.sync_copy(pos_v, pos_hbm.at[pl.ds(out_base, _BPW)])
    pltpu.sync_copy(neg_v, neg_hbm.at[pl.ds(out_base, _BPW)])


@jax.jit
def _run(uidx, pidx, nidx, user_table, item_table):
    mesh = plsc.VectorSubcoreMesh(core_axis_name="c", subcore_axis_name="s")
    f32 = jnp.float32
    kern = functools.partial(
        pl.kernel,
        out_type=[jax.ShapeDtypeStruct((_BATCH,), f32),
                  jax.ShapeDtypeStruct((_BATCH,), f32)],
        mesh=mesh,
        scratch_types=[
            pltpu.VMEM((_NCHUNK, _CHUNK), jnp.int32),
            pltpu.VMEM((_NCHUNK, _CHUNK), jnp.int32),
            pltpu.VMEM((_NCHUNK, _CHUNK), jnp.int32),
            pltpu.VMEM((_BPW, _EMBED), f32),
            pltpu.VMEM((_BPW, _EMBED), f32),
            pltpu.VMEM((_BPW, _EMBED), f32),
            pltpu.VMEM((_BPW,), f32),
            pltpu.VMEM((_BPW,), f32),
            pltpu.SemaphoreType.DMA,
        ],
        compiler_params=pltpu.CompilerParams(
            needs_layout_passes=False, use_tc_tiling_on_sc=False),
    )(_sc_body)
    return kern(uidx, pidx, nidx, user_table, item_table)


def kernel(user_inputs, pos_item_inputs, neg_item_inputs, user_table, item_table):
    uidx = jnp.asarray(user_inputs, jnp.int32).reshape(_NW * _NCHUNK, _CHUNK)
    pidx = jnp.asarray(pos_item_inputs, jnp.int32).reshape(_NW * _NCHUNK, _CHUNK)
    nidx = jnp.asarray(neg_item_inputs, jnp.int32).reshape(_NW * _NCHUNK, _CHUNK)
    pos, neg = _run(uidx, pidx, nidx, user_table, item_table)
    return pos, neg
